# baseline (speedup 1.0000x reference)
import functools

import jax
import jax.numpy as jnp
from jax import lax
from jax.experimental import pallas as pl
from jax.experimental.pallas import tpu as pltpu
from jax.experimental.pallas import tpu_sc as plsc

_PATTERNS = [
    [(i * 7) % 100, (i * 7 + 13) % 100, (i * 7 + 29) % 100, (i * 7 + 53) % 100]
    for i in range(26)
]
_B = 16384
_F = 100
_R = _B // 4


@functools.lru_cache(maxsize=None)
def _build_sc_match():
    info = plsc.get_sparse_core_info()
    nc, ns, lanes = info.num_cores, info.num_subcores, info.num_lanes
    nw = nc * ns
    wrows_per_w = _R // nw
    chunks = wrows_per_w // lanes
    rows_per_w = wrows_per_w * 4
    pitch = lanes + 1
    tsize = _F * pitch
    mesh = plsc.VectorSubcoreMesh(core_axis_name="c", subcore_axis_name="s")

    @functools.partial(
        pl.kernel,
        mesh=mesh,
        out_type=jax.ShapeDtypeStruct((_B,), jnp.int32),
        scratch_types=[
            pltpu.VMEM((wrows_per_w, _F), jnp.int32),
            pltpu.VMEM((rows_per_w,), jnp.int32),
            pltpu.VMEM((2 * tsize,), jnp.int32),
        ],
        compiler_params=pltpu.CompilerParams(needs_layout_passes=False),
    )
    def sc_match(in_hbm, out_hbm, wbuf, obuf, tbuf):
        wid = lax.axis_index("s") * nc + lax.axis_index("c")
        wrow0 = wid * wrows_per_w
        words_hbm = in_hbm.bitcast(jnp.int32)
        pltpu.sync_copy(words_hbm.at[pl.ds(wrow0, wrows_per_w), :], wbuf)

        lane = lax.iota(jnp.int32, lanes)
        lane_p = lane * pitch
        lane4 = lane * 4
        cols = sorted({col for pat in _PATTERNS for col in pat})
        starts = list(range(0, _F - lanes, lanes)) + [_F - lanes]

        def do_chunk(c, tbase):
            for l in range(lanes):
                r = c * lanes + l
                for g0 in starts:
                    v = wbuf[r, pl.ds(g0, lanes)]
                    idx = lane_p + (g0 * pitch + l + tbase)
                    plsc.store_scatter(tbuf, [idx], v)
            vals = {
                col: tbuf[pl.ds(col * pitch + tbase, lanes)] for col in cols
            }
            acc = None
            for p0, p1, p2, p3 in _PATTERNS:
                m = vals[p0] & vals[p1] & vals[p2] & vals[p3]
                acc = m if acc is None else (acc | m)
            for k in range(4):
                rk = lax.shift_right_logical(acc, 8 * k) & 1
                plsc.store_scatter(obuf, [lane4 + (c * lanes * 4 + k)], rk)

        for c in range(chunks):
            do_chunk(c, (c % 2) * tsize)
        pltpu.sync_copy(obuf, out_hbm.at[pl.ds(wid * rows_per_w, rows_per_w)])

    return sc_match


def kernel(inputs):
    out = _build_sc_match()(inputs.astype(jnp.uint8))
    return out != 0


# speedup vs baseline: 1.1584x; 1.1584x over previous
import functools

import jax
import jax.numpy as jnp
from jax import lax
from jax.experimental import pallas as pl
from jax.experimental.pallas import tpu as pltpu
from jax.experimental.pallas import tpu_sc as plsc

_PATTERNS = [
    [(i * 7) % 100, (i * 7 + 13) % 100, (i * 7 + 29) % 100, (i * 7 + 53) % 100]
    for i in range(26)
]
_B = 16384
_F = 100
_R = _B // 4


@functools.lru_cache(maxsize=None)
def _build_sc_match():
    info = plsc.get_sparse_core_info()
    nc, ns, lanes = info.num_cores, info.num_subcores, info.num_lanes
    nw = nc * ns
    wrows_per_w = _R // nw
    chunks = wrows_per_w // lanes
    rows_per_w = wrows_per_w * 4
    pitch = wrows_per_w + 1
    tsize = _F * pitch
    mesh = plsc.VectorSubcoreMesh(core_axis_name="c", subcore_axis_name="s")

    @functools.partial(
        pl.kernel,
        mesh=mesh,
        out_type=jax.ShapeDtypeStruct((_B,), jnp.int32),
        scratch_types=[
            pltpu.VMEM((wrows_per_w, _F), jnp.int32),
            pltpu.VMEM((rows_per_w,), jnp.int32),
            pltpu.VMEM((tsize,), jnp.int32),
        ],
        compiler_params=pltpu.CompilerParams(needs_layout_passes=False),
    )
    def sc_match(in_hbm, out_hbm, wbuf, obuf, tbuf):
        wid = lax.axis_index("s") * nc + lax.axis_index("c")
        wrow0 = wid * wrows_per_w
        words_hbm = in_hbm.bitcast(jnp.int32)
        pltpu.sync_copy(words_hbm.at[pl.ds(wrow0, wrows_per_w), :], wbuf)

        lane = lax.iota(jnp.int32, lanes)
        lane_p = lane * pitch
        lane4 = lane * 4
        cols = sorted({col for pat in _PATTERNS for col in pat})
        starts = list(range(0, _F - lanes, lanes)) + [_F - lanes]

        def tbody(c, carry):
            for l in range(lanes):
                r = c * lanes + l
                for g0 in starts:
                    v = wbuf[r, pl.ds(g0, lanes)]
                    idx = lane_p + (g0 * pitch + r)
                    plsc.store_scatter(tbuf, [idx], v)
            return carry

        lax.fori_loop(0, chunks, tbody, None)

        def ebody(c, carry):
            vals = {
                col: tbuf[pl.ds(col * pitch + c * lanes, lanes)]
                for col in cols
            }
            acc = None
            for p0, p1, p2, p3 in _PATTERNS:
                m = vals[p0] & vals[p1] & vals[p2] & vals[p3]
                acc = m if acc is None else (acc | m)
            for k in range(4):
                rk = lax.shift_right_logical(acc, 8 * k) & 1
                plsc.store_scatter(obuf, [lane4 + (c * lanes * 4 + k)], rk)
            return carry

        lax.fori_loop(0, chunks, ebody, None)
        pltpu.sync_copy(obuf, out_hbm.at[pl.ds(wid * rows_per_w, rows_per_w)])

    return sc_match


def kernel(inputs):
    out = _build_sc_match()(inputs.astype(jnp.uint8))
    return out != 0
